# baseline (speedup 1.0000x reference)
import jax
import jax.numpy as jnp
from jax import lax
from jax.experimental import pallas as pl
from jax.experimental.pallas import tpu as pltpu
from jax.experimental.pallas import tpu_sc as plsc

_N = 10000
_E = 320000
_H = 8
_D = 16
_HD = _H * _D
_HH = _HD // 2

_NC = 2
_NS = 16
_EPT = _E // _NS
_C = 80
_NCHUNK = _EPT // _C
_LAST = _NCHUNK - 1
_RPS = _N // _NS


def _sc_attn_body(src2d, dst2d, qtabs, kvtabs, pes,
                  eout3, accw_out, accz_out,
                  src3, dst3, q3, kv3, pe3, zc3,
                  accw, accz,
                  gsem0, gsem1, gsem2, wsem0, wsem1, wsem2, isem):
    c = lax.axis_index("c")
    s = lax.axis_index("s")
    ebase = s * _EPT
    rowbase = s * (_EPT // _C)
    gsem = (gsem0, gsem1, gsem2)
    wsem = (wsem0, wsem1, wsem2)
    qtab = qtabs.at[c]
    kvtab = kvtabs.at[c]

    zeros16 = jnp.zeros((16,), jnp.float32)

    def zrow(r, carry):
        for cc in range(_HH // 16):
            pe3[0, r, pl.ds(cc * 16, 16)] = zeros16
        zc3[0, r, pl.ds(0, 16)] = zeros16
        return carry

    lax.fori_loop(0, 25, zrow, 0)

    def zacc(k, carry):
        off = s * _RPS + k * 25
        pltpu.sync_copy(pe3.at[0, pl.ds(0, 25)], accw.at[pl.ds(off, 25)])
        pltpu.sync_copy(zc3.at[0, pl.ds(0, 25)], accz.at[pl.ds(off, 25)])
        return carry

    lax.fori_loop(0, 25, zacc, 0)
    plsc.subcore_barrier()

    lane = lax.iota(jnp.int32, 16)

    def idx_refs(g, sl):
        return ((src2d.at[pl.ds(rowbase + g, 1)], src3.at[pl.ds(sl, 1)]),
                (dst2d.at[pl.ds(rowbase + g, 1)], dst3.at[pl.ds(sl, 1)]))

    def gather_refs(g, sl):
        base = ebase + g * _C
        return ((qtab.at[dst3.at[sl]], q3.at[sl]),
                (kvtab.at[src3.at[sl]], kv3.at[sl]),
                (pes.at[c, pl.ds(base, _C)], pe3.at[sl]))

    def write_refs(g, sl):
        base = ebase + g * _C
        return ((pe3.at[sl], eout3.at[pl.ds(base, _C), c]),)

    def issue_idx(g, sl):
        for a, b in idx_refs(g, sl):
            pltpu.async_copy(a, b, isem)

    def drain_idx(g, sl):
        for a, b in idx_refs(g, sl):
            pltpu.make_async_copy(a, b, isem).wait()

    def issue_gathers(g, sl):
        for a, b in gather_refs(g, sl):
            pltpu.async_copy(a, b, gsem[sl])

    def drain_gathers(g, sl):
        for a, b in gather_refs(g, sl):
            pltpu.make_async_copy(a, b, gsem[sl]).wait()

    def issue_writes(g, sl):
        for a, b in write_refs(g, sl):
            pltpu.async_copy(a, b, wsem[sl])
        pltpu.sync_copy(q3.at[sl], accw.at[dst3.at[sl]], add=True)
        pltpu.sync_copy(zc3.at[sl], accz.at[dst3.at[sl]], add=True)

    def drain_writes(g, sl):
        for a, b in write_refs(g, sl):
            pltpu.make_async_copy(a, b, wsem[sl]).wait()

    def compute(sl):
        @plsc.parallel_loop(0, _C, unroll=2)
        def edge(j):
            zvec = jnp.zeros((16,), jnp.float32)
            for hl in range(4):
                o = hl * 16
                qv = q3[sl, j, pl.ds(o, 16)]
                kvv = kv3[sl, j, pl.ds(o, 16)]
                vv = kv3[sl, j, pl.ds(_HH + o, 16)]
                pev = pe3[sl, j, pl.ds(o, 16)]
                score = (kvv * qv) * pev
                pe3[sl, j, pl.ds(o, 16)] = score
                t = jnp.broadcast_to(jnp.sum(score), (16,))
                sv = jnp.exp(jnp.clip(t, -5.0, 5.0))
                q3[sl, j, pl.ds(o, 16)] = vv * sv
                zvec = jnp.where(lane == hl, sv, zvec)
            zc3[sl, j, pl.ds(0, 16)] = zvec

    issue_idx(0, 0)
    drain_idx(0, 0)
    issue_idx(1, 1)
    drain_idx(1, 1)
    issue_gathers(0, 0)

    def step(g, sl):
        drain_gathers(g, sl)
        compute(sl)

        @pl.when(g >= 1)
        def _():
            drain_writes(g - 1, (sl + 2) % 3)

        issue_writes(g, sl)

        @pl.when(g <= _LAST - 1)
        def _():
            @pl.when(g >= 1)
            def _():
                drain_idx(g + 1, (sl + 1) % 3)

            issue_gathers(g + 1, (sl + 1) % 3)

        @pl.when(g <= _LAST - 2)
        def _():
            issue_idx(g + 2, (sl + 2) % 3)

    def triple(i, carry):
        g0 = i * 3
        step(g0, 0)
        step(g0 + 1, 1)
        step(g0 + 2, 2)
        return carry

    lax.fori_loop(0, _NCHUNK // 3, triple, 0)
    step(_NCHUNK - 1, (_NCHUNK - 1) % 3)
    drain_writes(_NCHUNK - 1, (_NCHUNK - 1) % 3)

    plsc.subcore_barrier()
    pltpu.sync_copy(accw.at[pl.ds(s * _RPS, _RPS)],
                    accw_out.at[c, pl.ds(s * _RPS, _RPS)])
    pltpu.sync_copy(accz.at[pl.ds(s * _RPS, _RPS)],
                    accz_out.at[c, pl.ds(s * _RPS, _RPS)])


_sc_attn = pl.kernel(
    _sc_attn_body,
    out_type=[
        jax.ShapeDtypeStruct((_E, _NC, _HH), jnp.float32),
        jax.ShapeDtypeStruct((_NC, _N, _HH), jnp.float32),
        jax.ShapeDtypeStruct((_NC, _N, 16), jnp.float32),
    ],
    mesh=plsc.VectorSubcoreMesh(core_axis_name="c", subcore_axis_name="s"),
    compiler_params=pltpu.CompilerParams(use_tc_tiling_on_sc=False,
                                         needs_layout_passes=False),
    scratch_types=[
        pltpu.VMEM((3, _C), jnp.int32),
        pltpu.VMEM((3, _C), jnp.int32),
        pltpu.VMEM((3, _C, _HH), jnp.float32),
        pltpu.VMEM((3, _C, 2 * _HH), jnp.float32),
        pltpu.VMEM((3, _C, _HH), jnp.float32),
        pltpu.VMEM((3, _C, 16), jnp.float32),
        pltpu.VMEM_SHARED((_N, _HH), jnp.float32),
        pltpu.VMEM_SHARED((_N, 16), jnp.float32),
        pltpu.SemaphoreType.DMA,
        pltpu.SemaphoreType.DMA,
        pltpu.SemaphoreType.DMA,
        pltpu.SemaphoreType.DMA,
        pltpu.SemaphoreType.DMA,
        pltpu.SemaphoreType.DMA,
        pltpu.SemaphoreType.DMA,
    ],
)


_TB = 2000


def _tables_body(h_ref, wq_ref, bq_ref, wkv_ref, bkv_ref, q_out, kv_out):
    hb = h_ref[...]
    q_out[0] = jnp.dot(hb, wq_ref[0], preferred_element_type=jnp.float32,
                       precision=lax.Precision.HIGHEST) + bq_ref[0]
    kv_out[0] = jnp.dot(hb, wkv_ref[0], preferred_element_type=jnp.float32,
                        precision=lax.Precision.HIGHEST) + bkv_ref[0]


_tables = pl.pallas_call(
    _tables_body,
    grid=(_NC, _N // _TB),
    in_specs=[
        pl.BlockSpec((_TB, _HD), lambda cc, i: (i, 0)),
        pl.BlockSpec((1, _HD, _HH), lambda cc, i: (cc, 0, 0)),
        pl.BlockSpec((1, 1, _HH), lambda cc, i: (cc, 0, 0)),
        pl.BlockSpec((1, _HD, 2 * _HH), lambda cc, i: (cc, 0, 0)),
        pl.BlockSpec((1, 1, 2 * _HH), lambda cc, i: (cc, 0, 0)),
    ],
    out_specs=[
        pl.BlockSpec((1, _TB, _HH), lambda cc, i: (cc, i, 0)),
        pl.BlockSpec((1, _TB, 2 * _HH), lambda cc, i: (cc, i, 0)),
    ],
    out_shape=[
        jax.ShapeDtypeStruct((_NC, _N, _HH), jnp.float32),
        jax.ShapeDtypeStruct((_NC, _N, 2 * _HH), jnp.float32),
    ],
)

_EB = 2000


def _pe_body(e_ref, we_ref, be_ref, out_ref):
    out_ref[0] = (jnp.dot(e_ref[...], we_ref[0],
                          preferred_element_type=jnp.float32,
                          precision=lax.Precision.HIGHEST) * 0.25
                  + be_ref[0] * 0.25)


_pe = pl.pallas_call(
    _pe_body,
    grid=(_NC, _E // _EB),
    in_specs=[
        pl.BlockSpec((_EB, _HD), lambda cc, i: (i, 0)),
        pl.BlockSpec((1, _HD, _HH), lambda cc, i: (cc, 0, 0)),
        pl.BlockSpec((1, 1, _HH), lambda cc, i: (cc, 0, 0)),
    ],
    out_specs=pl.BlockSpec((1, _EB, _HH), lambda cc, i: (cc, i, 0)),
    out_shape=jax.ShapeDtypeStruct((_NC, _E, _HH), jnp.float32),
)

_CB = 2000


def _combine_body(a0_ref, a1_ref, z0_ref, z1_ref, r_ref, o0_ref, o1_ref):
    zr0 = lax.dot_general(z0_ref[...], r_ref[...], (((1,), (0,)), ((), ())),
                          precision=lax.Precision.HIGHEST,
                          preferred_element_type=jnp.float32)
    zr1 = lax.dot_general(z1_ref[...], r_ref[...], (((1,), (0,)), ((), ())),
                          precision=lax.Precision.HIGHEST,
                          preferred_element_type=jnp.float32)
    o0_ref[...] = a0_ref[...] / (zr0 + 1e-6)
    o1_ref[...] = a1_ref[...] / (zr1 + 1e-6)


_combine = pl.pallas_call(
    _combine_body,
    grid=(_N // _CB,),
    in_specs=[
        pl.BlockSpec((_CB, _HH), lambda i: (i, 0)),
        pl.BlockSpec((_CB, _HH), lambda i: (i, 0)),
        pl.BlockSpec((_CB, 16), lambda i: (i, 0)),
        pl.BlockSpec((_CB, 16), lambda i: (i, 0)),
        pl.BlockSpec((16, _HH), lambda i: (0, 0)),
    ],
    out_specs=[
        pl.BlockSpec((_CB, _HH), lambda i: (i, 0)),
        pl.BlockSpec((_CB, _HH), lambda i: (i, 0)),
    ],
    out_shape=[
        jax.ShapeDtypeStruct((_N, _HH), jnp.float32),
        jax.ShapeDtypeStruct((_N, _HH), jnp.float32),
    ],
)


def kernel(edge_index, h, e, Wq, bq, Wk, bk, Wv, bv, We, be):
    src = edge_index[0].astype(jnp.int32)
    dst = edge_index[1].astype(jnp.int32)
    src2d = src.reshape(_E // _C, _C)
    dst2d = dst.reshape(_E // _C, _C)
    Wqs = jnp.stack([Wq[:, :_HH], Wq[:, _HH:]])
    bqs = jnp.stack([bq[:_HH], bq[_HH:]]).reshape(_NC, 1, _HH)
    Wkvs = jnp.stack([
        jnp.concatenate([Wk[:, :_HH], Wv[:, :_HH]], axis=1),
        jnp.concatenate([Wk[:, _HH:], Wv[:, _HH:]], axis=1),
    ])
    bkvs = jnp.stack([
        jnp.concatenate([bk[:_HH], bv[:_HH]]),
        jnp.concatenate([bk[_HH:], bv[_HH:]]),
    ]).reshape(_NC, 1, 2 * _HH)
    Wes = jnp.stack([We[:, :_HH], We[:, _HH:]])
    bes = jnp.stack([be[:_HH], be[_HH:]]).reshape(_NC, 1, _HH)

    qtabs, kvtabs = _tables(h, Wqs, bqs, Wkvs, bkvs)
    pes = _pe(e, Wes, bes)
    eout3, accw, accz = _sc_attn(src2d, dst2d, qtabs, kvtabs, pes)
    ra = jnp.concatenate([
        jnp.repeat(jnp.eye(4, dtype=jnp.float32), _D, axis=1),
        jnp.zeros((12, _HH), jnp.float32),
    ], axis=0)
    h0, h1 = _combine(accw[0], accw[1], accz[0], accz[1], ra)
    hout = jnp.concatenate([h0.reshape(_N, 4, _D), h1.reshape(_N, 4, _D)],
                           axis=1)
    return hout, eout3.reshape(_E, _H, _D)


# speedup vs baseline: 2.1920x; 2.1920x over previous
import jax
import jax.numpy as jnp
from jax import lax
from jax.experimental import pallas as pl
from jax.experimental.pallas import tpu as pltpu
from jax.experimental.pallas import tpu_sc as plsc

_N = 10000
_E = 320000
_H = 8
_D = 16
_HD = _H * _D
_HH = _HD // 2

_NC = 2
_NS = 16
_EPT = _E // _NS
_C = 80
_NCHUNK = _EPT // _C
_LAST = _NCHUNK - 1
_RPS = _N // _NS


def _sc_attn_body(src2d, dst2d, qtabs, kvtabs, pes,
                  eout3, accw_out, accz_out,
                  src3, dst3, q3, kv3, pe3, zc3,
                  accw, accz,
                  gsem0, gsem1, gsem2, wsem0, wsem1, wsem2, isem):
    c = lax.axis_index("c")
    s = lax.axis_index("s")
    ebase = s * _EPT
    rowbase = s * (_EPT // _C)
    gsem = (gsem0, gsem1, gsem2)
    wsem = (wsem0, wsem1, wsem2)
    qtab = qtabs.at[c]
    kvtab = kvtabs.at[c]
    cw = pl.multiple_of(c * _HH, _HH)

    zeros16 = jnp.zeros((16,), jnp.float32)

    def zrow(r, carry):
        for cc in range(_HH // 16):
            pe3[0, r, pl.ds(cc * 16, 16)] = zeros16
        zc3[0, r, pl.ds(0, 16)] = zeros16
        return carry

    lax.fori_loop(0, 25, zrow, 0)

    def zacc(k, carry):
        off = s * _RPS + k * 25
        pltpu.sync_copy(pe3.at[0, pl.ds(0, 25)], accw.at[pl.ds(off, 25)])
        pltpu.sync_copy(zc3.at[0, pl.ds(0, 25)], accz.at[pl.ds(off, 25)])
        return carry

    lax.fori_loop(0, 25, zacc, 0)
    plsc.subcore_barrier()

    lane = lax.iota(jnp.int32, 16)

    def idx_refs(g, sl):
        return ((src2d.at[pl.ds(rowbase + g, 1)], src3.at[pl.ds(sl, 1)]),
                (dst2d.at[pl.ds(rowbase + g, 1)], dst3.at[pl.ds(sl, 1)]))

    def gather_refs(g, sl):
        base = ebase + g * _C
        return ((qtab.at[dst3.at[sl]], q3.at[sl]),
                (kvtab.at[src3.at[sl]], kv3.at[sl]),
                (pes.at[pl.ds(base, _C), pl.ds(cw, _HH)], pe3.at[sl]))

    def write_refs(g, sl):
        base = ebase + g * _C
        return ((pe3.at[sl], eout3.at[pl.ds(base, _C), pl.ds(cw, _HH)]),)

    def issue_idx(g, sl):
        for a, b in idx_refs(g, sl):
            pltpu.async_copy(a, b, isem)

    def drain_idx(g, sl):
        for a, b in idx_refs(g, sl):
            pltpu.make_async_copy(a, b, isem).wait()

    def issue_gathers(g, sl):
        for a, b in gather_refs(g, sl):
            pltpu.async_copy(a, b, gsem[sl])

    def drain_gathers(g, sl):
        for a, b in gather_refs(g, sl):
            pltpu.make_async_copy(a, b, gsem[sl]).wait()

    def issue_writes(g, sl):
        for a, b in write_refs(g, sl):
            pltpu.async_copy(a, b, wsem[sl])
        pltpu.sync_copy(q3.at[sl], accw.at[dst3.at[sl]], add=True)
        pltpu.sync_copy(zc3.at[sl], accz.at[dst3.at[sl]], add=True)

    def drain_writes(g, sl):
        for a, b in write_refs(g, sl):
            pltpu.make_async_copy(a, b, wsem[sl]).wait()

    def compute(sl):
        @plsc.parallel_loop(0, _C, unroll=2)
        def edge(j):
            zvec = jnp.zeros((16,), jnp.float32)
            for hl in range(4):
                o = hl * 16
                qv = q3[sl, j, pl.ds(o, 16)]
                kvv = kv3[sl, j, pl.ds(o, 16)]
                vv = kv3[sl, j, pl.ds(_HH + o, 16)]
                pev = pe3[sl, j, pl.ds(o, 16)]
                score = (kvv * qv) * pev
                pe3[sl, j, pl.ds(o, 16)] = score
                t = jnp.broadcast_to(jnp.sum(score), (16,))
                sv = jnp.exp(jnp.clip(t, -5.0, 5.0))
                q3[sl, j, pl.ds(o, 16)] = vv * sv
                zvec = jnp.where(lane == hl, sv, zvec)
            zc3[sl, j, pl.ds(0, 16)] = zvec

    issue_idx(0, 0)
    drain_idx(0, 0)
    issue_idx(1, 1)
    drain_idx(1, 1)
    issue_gathers(0, 0)

    def step(g, sl):
        drain_gathers(g, sl)
        compute(sl)

        @pl.when(g >= 1)
        def _():
            drain_writes(g - 1, (sl + 2) % 3)

        issue_writes(g, sl)

        @pl.when(g <= _LAST - 1)
        def _():
            @pl.when(g >= 1)
            def _():
                drain_idx(g + 1, (sl + 1) % 3)

            issue_gathers(g + 1, (sl + 1) % 3)

        @pl.when(g <= _LAST - 2)
        def _():
            issue_idx(g + 2, (sl + 2) % 3)

    def triple(i, carry):
        g0 = i * 3
        step(g0, 0)
        step(g0 + 1, 1)
        step(g0 + 2, 2)
        return carry

    lax.fori_loop(0, _NCHUNK // 3, triple, 0)
    step(_NCHUNK - 1, (_NCHUNK - 1) % 3)
    drain_writes(_NCHUNK - 1, (_NCHUNK - 1) % 3)

    plsc.subcore_barrier()
    pltpu.sync_copy(accw.at[pl.ds(s * _RPS, _RPS)],
                    accw_out.at[c, pl.ds(s * _RPS, _RPS)])
    pltpu.sync_copy(accz.at[pl.ds(s * _RPS, _RPS)],
                    accz_out.at[c, pl.ds(s * _RPS, _RPS)])


_sc_attn = pl.kernel(
    _sc_attn_body,
    out_type=[
        jax.ShapeDtypeStruct((_E, _HD), jnp.float32),
        jax.ShapeDtypeStruct((_NC, _N, _HH), jnp.float32),
        jax.ShapeDtypeStruct((_NC, _N, 16), jnp.float32),
    ],
    mesh=plsc.VectorSubcoreMesh(core_axis_name="c", subcore_axis_name="s"),
    compiler_params=pltpu.CompilerParams(use_tc_tiling_on_sc=False,
                                         needs_layout_passes=False),
    scratch_types=[
        pltpu.VMEM((3, _C), jnp.int32),
        pltpu.VMEM((3, _C), jnp.int32),
        pltpu.VMEM((3, _C, _HH), jnp.float32),
        pltpu.VMEM((3, _C, 2 * _HH), jnp.float32),
        pltpu.VMEM((3, _C, _HH), jnp.float32),
        pltpu.VMEM((3, _C, 16), jnp.float32),
        pltpu.VMEM_SHARED((_N, _HH), jnp.float32),
        pltpu.VMEM_SHARED((_N, 16), jnp.float32),
        pltpu.SemaphoreType.DMA,
        pltpu.SemaphoreType.DMA,
        pltpu.SemaphoreType.DMA,
        pltpu.SemaphoreType.DMA,
        pltpu.SemaphoreType.DMA,
        pltpu.SemaphoreType.DMA,
        pltpu.SemaphoreType.DMA,
    ],
)


_TB = 2000


def _tables_body(h_ref, wq_ref, bq_ref, wkv_ref, bkv_ref, q_out, kv_out):
    hb = h_ref[...]
    q_out[0] = jnp.dot(hb, wq_ref[0], preferred_element_type=jnp.float32,
                       precision=lax.Precision.HIGHEST) + bq_ref[0]
    kv_out[0] = jnp.dot(hb, wkv_ref[0], preferred_element_type=jnp.float32,
                        precision=lax.Precision.HIGHEST) + bkv_ref[0]


_tables = pl.pallas_call(
    _tables_body,
    grid=(_NC, _N // _TB),
    in_specs=[
        pl.BlockSpec((_TB, _HD), lambda cc, i: (i, 0)),
        pl.BlockSpec((1, _HD, _HH), lambda cc, i: (cc, 0, 0)),
        pl.BlockSpec((1, 1, _HH), lambda cc, i: (cc, 0, 0)),
        pl.BlockSpec((1, _HD, 2 * _HH), lambda cc, i: (cc, 0, 0)),
        pl.BlockSpec((1, 1, 2 * _HH), lambda cc, i: (cc, 0, 0)),
    ],
    out_specs=[
        pl.BlockSpec((1, _TB, _HH), lambda cc, i: (cc, i, 0)),
        pl.BlockSpec((1, _TB, 2 * _HH), lambda cc, i: (cc, i, 0)),
    ],
    out_shape=[
        jax.ShapeDtypeStruct((_NC, _N, _HH), jnp.float32),
        jax.ShapeDtypeStruct((_NC, _N, 2 * _HH), jnp.float32),
    ],
)

_EB = 2000


def _pe_body(e_ref, we_ref, be_ref, out_ref):
    out_ref[...] = (jnp.dot(e_ref[...], we_ref[...],
                            preferred_element_type=jnp.float32) * 0.25
                    + be_ref[...] * 0.25)


_pe = pl.pallas_call(
    _pe_body,
    grid=(_E // _EB,),
    in_specs=[
        pl.BlockSpec((_EB, _HD), lambda i: (i, 0)),
        pl.BlockSpec((_HD, _HD), lambda i: (0, 0)),
        pl.BlockSpec((1, _HD), lambda i: (0, 0)),
    ],
    out_specs=pl.BlockSpec((_EB, _HD), lambda i: (i, 0)),
    out_shape=jax.ShapeDtypeStruct((_E, _HD), jnp.float32),
)

_CB = 2000


def _combine_body(a0_ref, a1_ref, z0_ref, z1_ref, r_ref, o0_ref, o1_ref):
    zr0 = lax.dot_general(z0_ref[...], r_ref[...], (((1,), (0,)), ((), ())),
                          precision=lax.Precision.HIGHEST,
                          preferred_element_type=jnp.float32)
    zr1 = lax.dot_general(z1_ref[...], r_ref[...], (((1,), (0,)), ((), ())),
                          precision=lax.Precision.HIGHEST,
                          preferred_element_type=jnp.float32)
    o0_ref[...] = a0_ref[...] / (zr0 + 1e-6)
    o1_ref[...] = a1_ref[...] / (zr1 + 1e-6)


_combine = pl.pallas_call(
    _combine_body,
    grid=(_N // _CB,),
    in_specs=[
        pl.BlockSpec((_CB, _HH), lambda i: (i, 0)),
        pl.BlockSpec((_CB, _HH), lambda i: (i, 0)),
        pl.BlockSpec((_CB, 16), lambda i: (i, 0)),
        pl.BlockSpec((_CB, 16), lambda i: (i, 0)),
        pl.BlockSpec((16, _HH), lambda i: (0, 0)),
    ],
    out_specs=[
        pl.BlockSpec((_CB, _HH), lambda i: (i, 0)),
        pl.BlockSpec((_CB, _HH), lambda i: (i, 0)),
    ],
    out_shape=[
        jax.ShapeDtypeStruct((_N, _HH), jnp.float32),
        jax.ShapeDtypeStruct((_N, _HH), jnp.float32),
    ],
)


def kernel(edge_index, h, e, Wq, bq, Wk, bk, Wv, bv, We, be):
    src = edge_index[0].astype(jnp.int32)
    dst = edge_index[1].astype(jnp.int32)
    src2d = src.reshape(_E // _C, _C)
    dst2d = dst.reshape(_E // _C, _C)
    Wqs = jnp.stack([Wq[:, :_HH], Wq[:, _HH:]])
    bqs = jnp.stack([bq[:_HH], bq[_HH:]]).reshape(_NC, 1, _HH)
    Wkvs = jnp.stack([
        jnp.concatenate([Wk[:, :_HH], Wv[:, :_HH]], axis=1),
        jnp.concatenate([Wk[:, _HH:], Wv[:, _HH:]], axis=1),
    ])
    bkvs = jnp.stack([
        jnp.concatenate([bk[:_HH], bv[:_HH]]),
        jnp.concatenate([bk[_HH:], bv[_HH:]]),
    ]).reshape(_NC, 1, 2 * _HH)
    qtabs, kvtabs = _tables(h, Wqs, bqs, Wkvs, bkvs)
    pes = _pe(e, We, be.reshape(1, -1))
    eout3, accw, accz = _sc_attn(src2d, dst2d, qtabs, kvtabs, pes)
    ra = jnp.concatenate([
        jnp.repeat(jnp.eye(4, dtype=jnp.float32), _D, axis=1),
        jnp.zeros((12, _HH), jnp.float32),
    ], axis=0)
    h0, h1 = _combine(accw[0], accw[1], accz[0], accz[1], ra)
    hout = jnp.concatenate([h0.reshape(_N, 4, _D), h1.reshape(_N, 4, _D)],
                           axis=1)
    return hout, eout3.reshape(_E, _H, _D)


# speedup vs baseline: 2.2051x; 1.0059x over previous
import jax
import jax.numpy as jnp
from jax import lax
from jax.experimental import pallas as pl
from jax.experimental.pallas import tpu as pltpu
from jax.experimental.pallas import tpu_sc as plsc

_N = 10000
_E = 320000
_H = 8
_D = 16
_HD = _H * _D
_HH = _HD // 2

_NC = 2
_NS = 16
_EPT = _E // _NS
_C = 80
_NCHUNK = _EPT // _C
_LAST = _NCHUNK - 1
_RPS = _N // _NS


def _sc_attn_body(src2d, dst2d, qtabs, kvtabs, pes,
                  eout3, accw_out, accz_out,
                  src3, dst3, q3, kv3, pe3, zc3,
                  accw, accz,
                  gsem0, gsem1, gsem2, wsem0, wsem1, wsem2, isem):
    c = lax.axis_index("c")
    s = lax.axis_index("s")
    ebase = s * _EPT
    rowbase = s * (_EPT // _C)
    gsem = (gsem0, gsem1, gsem2)
    wsem = (wsem0, wsem1, wsem2)
    qtab = qtabs.at[c]
    kvtab = kvtabs.at[c]
    cw = pl.multiple_of(c * _HH, _HH)

    zeros16 = jnp.zeros((16,), jnp.float32)

    def zrow(r, carry):
        for cc in range(_HH // 16):
            pe3[0, r, pl.ds(cc * 16, 16)] = zeros16
        zc3[0, r, pl.ds(0, 16)] = zeros16
        return carry

    lax.fori_loop(0, 25, zrow, 0)

    def zacc(k, carry):
        off = s * _RPS + k * 25
        pltpu.sync_copy(pe3.at[0, pl.ds(0, 25)], accw.at[pl.ds(off, 25)])
        pltpu.sync_copy(zc3.at[0, pl.ds(0, 25)], accz.at[pl.ds(off, 25)])
        return carry

    lax.fori_loop(0, 25, zacc, 0)
    plsc.subcore_barrier()

    lane = lax.iota(jnp.int32, 16)

    def idx_refs(g, sl):
        return ((src2d.at[pl.ds(rowbase + g, 1)], src3.at[pl.ds(sl, 1)]),
                (dst2d.at[pl.ds(rowbase + g, 1)], dst3.at[pl.ds(sl, 1)]))

    def gather_refs(g, sl):
        base = ebase + g * _C
        return ((qtab.at[dst3.at[sl]], q3.at[sl]),
                (kvtab.at[src3.at[sl]], kv3.at[sl]),
                (pes.at[pl.ds(base, _C), pl.ds(cw, _HH)], pe3.at[sl]))

    def write_refs(g, sl):
        base = ebase + g * _C
        return ((pe3.at[sl], eout3.at[pl.ds(base, _C), pl.ds(cw, _HH)]),)

    def issue_idx(g, sl):
        for a, b in idx_refs(g, sl):
            pltpu.async_copy(a, b, isem)

    def drain_idx(g, sl):
        for a, b in idx_refs(g, sl):
            pltpu.make_async_copy(a, b, isem).wait()

    def issue_gathers(g, sl):
        for a, b in gather_refs(g, sl):
            pltpu.async_copy(a, b, gsem[sl])

    def drain_gathers(g, sl):
        for a, b in gather_refs(g, sl):
            pltpu.make_async_copy(a, b, gsem[sl]).wait()

    def issue_writes(g, sl):
        for a, b in write_refs(g, sl):
            pltpu.async_copy(a, b, wsem[sl])
        pltpu.sync_copy(q3.at[sl], accw.at[dst3.at[sl]], add=True)
        pltpu.sync_copy(zc3.at[sl], accz.at[dst3.at[sl]], add=True)

    def drain_writes(g, sl):
        for a, b in write_refs(g, sl):
            pltpu.make_async_copy(a, b, wsem[sl]).wait()

    def compute(sl):
        @plsc.parallel_loop(0, _C, unroll=4)
        def edge(j):
            zvec = jnp.zeros((16,), jnp.float32)
            for hl in range(4):
                o = hl * 16
                qv = q3[sl, j, pl.ds(o, 16)]
                kvv = kv3[sl, j, pl.ds(o, 16)]
                vv = kv3[sl, j, pl.ds(_HH + o, 16)]
                pev = pe3[sl, j, pl.ds(o, 16)]
                score = (kvv * qv) * pev
                pe3[sl, j, pl.ds(o, 16)] = score
                t = jnp.broadcast_to(jnp.sum(score), (16,))
                sv = jnp.exp(jnp.clip(t, -5.0, 5.0))
                q3[sl, j, pl.ds(o, 16)] = vv * sv
                zvec = jnp.where(lane == hl, sv, zvec)
            zc3[sl, j, pl.ds(0, 16)] = zvec

    issue_idx(0, 0)
    drain_idx(0, 0)
    issue_idx(1, 1)
    drain_idx(1, 1)
    issue_gathers(0, 0)

    def step(g, sl):
        drain_gathers(g, sl)
        compute(sl)

        @pl.when(g >= 1)
        def _():
            drain_writes(g - 1, (sl + 2) % 3)

        issue_writes(g, sl)

        @pl.when(g <= _LAST - 1)
        def _():
            @pl.when(g >= 1)
            def _():
                drain_idx(g + 1, (sl + 1) % 3)

            issue_gathers(g + 1, (sl + 1) % 3)

        @pl.when(g <= _LAST - 2)
        def _():
            issue_idx(g + 2, (sl + 2) % 3)

    def triple(i, carry):
        g0 = i * 3
        step(g0, 0)
        step(g0 + 1, 1)
        step(g0 + 2, 2)
        return carry

    lax.fori_loop(0, _NCHUNK // 3, triple, 0)
    step(_NCHUNK - 1, (_NCHUNK - 1) % 3)
    drain_writes(_NCHUNK - 1, (_NCHUNK - 1) % 3)

    plsc.subcore_barrier()
    pltpu.sync_copy(accw.at[pl.ds(s * _RPS, _RPS)],
                    accw_out.at[c, pl.ds(s * _RPS, _RPS)])
    pltpu.sync_copy(accz.at[pl.ds(s * _RPS, _RPS)],
                    accz_out.at[c, pl.ds(s * _RPS, _RPS)])


_sc_attn = pl.kernel(
    _sc_attn_body,
    out_type=[
        jax.ShapeDtypeStruct((_E, _HD), jnp.float32),
        jax.ShapeDtypeStruct((_NC, _N, _HH), jnp.float32),
        jax.ShapeDtypeStruct((_NC, _N, 16), jnp.float32),
    ],
    mesh=plsc.VectorSubcoreMesh(core_axis_name="c", subcore_axis_name="s"),
    compiler_params=pltpu.CompilerParams(use_tc_tiling_on_sc=False,
                                         needs_layout_passes=False),
    scratch_types=[
        pltpu.VMEM((3, _C), jnp.int32),
        pltpu.VMEM((3, _C), jnp.int32),
        pltpu.VMEM((3, _C, _HH), jnp.float32),
        pltpu.VMEM((3, _C, 2 * _HH), jnp.float32),
        pltpu.VMEM((3, _C, _HH), jnp.float32),
        pltpu.VMEM((3, _C, 16), jnp.float32),
        pltpu.VMEM_SHARED((_N, _HH), jnp.float32),
        pltpu.VMEM_SHARED((_N, 16), jnp.float32),
        pltpu.SemaphoreType.DMA,
        pltpu.SemaphoreType.DMA,
        pltpu.SemaphoreType.DMA,
        pltpu.SemaphoreType.DMA,
        pltpu.SemaphoreType.DMA,
        pltpu.SemaphoreType.DMA,
        pltpu.SemaphoreType.DMA,
    ],
)


_TB = 2000


def _tables_body(h_ref, wq_ref, bq_ref, wkv_ref, bkv_ref, q_out, kv_out):
    hb = h_ref[...]
    q_out[0] = jnp.dot(hb, wq_ref[0], preferred_element_type=jnp.float32,
                       precision=lax.Precision.HIGHEST) + bq_ref[0]
    kv_out[0] = jnp.dot(hb, wkv_ref[0], preferred_element_type=jnp.float32,
                        precision=lax.Precision.HIGHEST) + bkv_ref[0]


_tables = pl.pallas_call(
    _tables_body,
    grid=(_NC, _N // _TB),
    in_specs=[
        pl.BlockSpec((_TB, _HD), lambda cc, i: (i, 0)),
        pl.BlockSpec((1, _HD, _HH), lambda cc, i: (cc, 0, 0)),
        pl.BlockSpec((1, 1, _HH), lambda cc, i: (cc, 0, 0)),
        pl.BlockSpec((1, _HD, 2 * _HH), lambda cc, i: (cc, 0, 0)),
        pl.BlockSpec((1, 1, 2 * _HH), lambda cc, i: (cc, 0, 0)),
    ],
    out_specs=[
        pl.BlockSpec((1, _TB, _HH), lambda cc, i: (cc, i, 0)),
        pl.BlockSpec((1, _TB, 2 * _HH), lambda cc, i: (cc, i, 0)),
    ],
    out_shape=[
        jax.ShapeDtypeStruct((_NC, _N, _HH), jnp.float32),
        jax.ShapeDtypeStruct((_NC, _N, 2 * _HH), jnp.float32),
    ],
)

_EB = 2000


def _pe_body(e_ref, we_ref, be_ref, out_ref):
    out_ref[...] = (jnp.dot(e_ref[...], we_ref[...],
                            preferred_element_type=jnp.float32) * 0.25
                    + be_ref[...] * 0.25)


_pe = pl.pallas_call(
    _pe_body,
    grid=(_E // _EB,),
    in_specs=[
        pl.BlockSpec((_EB, _HD), lambda i: (i, 0)),
        pl.BlockSpec((_HD, _HD), lambda i: (0, 0)),
        pl.BlockSpec((1, _HD), lambda i: (0, 0)),
    ],
    out_specs=pl.BlockSpec((_EB, _HD), lambda i: (i, 0)),
    out_shape=jax.ShapeDtypeStruct((_E, _HD), jnp.float32),
)

_CB = 2000


def _combine_body(a0_ref, a1_ref, z0_ref, z1_ref, r_ref, o0_ref, o1_ref):
    zr0 = lax.dot_general(z0_ref[...], r_ref[...], (((1,), (0,)), ((), ())),
                          precision=lax.Precision.HIGHEST,
                          preferred_element_type=jnp.float32)
    zr1 = lax.dot_general(z1_ref[...], r_ref[...], (((1,), (0,)), ((), ())),
                          precision=lax.Precision.HIGHEST,
                          preferred_element_type=jnp.float32)
    o0_ref[...] = a0_ref[...] / (zr0 + 1e-6)
    o1_ref[...] = a1_ref[...] / (zr1 + 1e-6)


_combine = pl.pallas_call(
    _combine_body,
    grid=(_N // _CB,),
    in_specs=[
        pl.BlockSpec((_CB, _HH), lambda i: (i, 0)),
        pl.BlockSpec((_CB, _HH), lambda i: (i, 0)),
        pl.BlockSpec((_CB, 16), lambda i: (i, 0)),
        pl.BlockSpec((_CB, 16), lambda i: (i, 0)),
        pl.BlockSpec((16, _HH), lambda i: (0, 0)),
    ],
    out_specs=[
        pl.BlockSpec((_CB, _HH), lambda i: (i, 0)),
        pl.BlockSpec((_CB, _HH), lambda i: (i, 0)),
    ],
    out_shape=[
        jax.ShapeDtypeStruct((_N, _HH), jnp.float32),
        jax.ShapeDtypeStruct((_N, _HH), jnp.float32),
    ],
)


def kernel(edge_index, h, e, Wq, bq, Wk, bk, Wv, bv, We, be):
    src = edge_index[0].astype(jnp.int32)
    dst = edge_index[1].astype(jnp.int32)
    src2d = src.reshape(_E // _C, _C)
    dst2d = dst.reshape(_E // _C, _C)
    Wqs = jnp.stack([Wq[:, :_HH], Wq[:, _HH:]])
    bqs = jnp.stack([bq[:_HH], bq[_HH:]]).reshape(_NC, 1, _HH)
    Wkvs = jnp.stack([
        jnp.concatenate([Wk[:, :_HH], Wv[:, :_HH]], axis=1),
        jnp.concatenate([Wk[:, _HH:], Wv[:, _HH:]], axis=1),
    ])
    bkvs = jnp.stack([
        jnp.concatenate([bk[:_HH], bv[:_HH]]),
        jnp.concatenate([bk[_HH:], bv[_HH:]]),
    ]).reshape(_NC, 1, 2 * _HH)
    qtabs, kvtabs = _tables(h, Wqs, bqs, Wkvs, bkvs)
    pes = _pe(e, We, be.reshape(1, -1))
    eout3, accw, accz = _sc_attn(src2d, dst2d, qtabs, kvtabs, pes)
    ra = jnp.concatenate([
        jnp.repeat(jnp.eye(4, dtype=jnp.float32), _D, axis=1),
        jnp.zeros((12, _HH), jnp.float32),
    ], axis=0)
    h0, h1 = _combine(accw[0], accw[1], accz[0], accz[1], ra)
    hout = jnp.concatenate([h0.reshape(_N, 4, _D), h1.reshape(_N, 4, _D)],
                           axis=1)
    return hout, eout3.reshape(_E, _H, _D)


# speedup vs baseline: 2.2674x; 1.0282x over previous
import jax
import jax.numpy as jnp
from jax import lax
from jax.experimental import pallas as pl
from jax.experimental.pallas import tpu as pltpu
from jax.experimental.pallas import tpu_sc as plsc

_N = 10000
_E = 320000
_H = 8
_D = 16
_HD = _H * _D
_HH = _HD // 2

_NC = 2
_NS = 16
_EPT = _E // _NS
_C = 80
_NCHUNK = _EPT // _C
_LAST = _NCHUNK - 1
_RPS = _N // _NS


_AW = _HH + 16


def _sc_attn_body(src2d, dst2d, qtabs, kvtabs, pes,
                  eout3, accw_out,
                  src3, dst3, q3, kv3, pe3, co3,
                  accw,
                  gsem0, gsem1, gsem2, wsem0, wsem1, wsem2, isem):
    c = lax.axis_index("c")
    s = lax.axis_index("s")
    ebase = s * _EPT
    rowbase = s * (_EPT // _C)
    gsem = (gsem0, gsem1, gsem2)
    wsem = (wsem0, wsem1, wsem2)
    qtab = qtabs.at[c]
    kvtab = kvtabs.at[c]
    cw = pl.multiple_of(c * _HH, _HH)

    zeros16 = jnp.zeros((16,), jnp.float32)

    def zrow(r, carry):
        for cc in range(_AW // 16):
            co3[r, pl.ds(cc * 16, 16)] = zeros16
        return carry

    lax.fori_loop(0, 25, zrow, 0)

    def zacc(k, carry):
        off = s * _RPS + k * 25
        pltpu.sync_copy(co3.at[pl.ds(0, 25)], accw.at[pl.ds(off, 25)])
        return carry

    lax.fori_loop(0, 25, zacc, 0)
    plsc.subcore_barrier()

    lane = lax.iota(jnp.int32, 16)

    def idx_refs(g, sl):
        return ((src2d.at[pl.ds(rowbase + g, 1)], src3.at[pl.ds(sl, 1)]),
                (dst2d.at[pl.ds(rowbase + g, 1)], dst3.at[pl.ds(sl, 1)]))

    def gather_refs(g, sl):
        base = ebase + g * _C
        return ((qtab.at[dst3.at[sl]], q3.at[sl]),
                (kvtab.at[src3.at[sl]], kv3.at[sl]),
                (pes.at[pl.ds(base, _C), pl.ds(cw, _HH)], pe3.at[sl]))

    def write_refs(g, sl):
        base = ebase + g * _C
        return ((pe3.at[sl], eout3.at[pl.ds(base, _C), pl.ds(cw, _HH)]),)

    def issue_idx(g, sl):
        for a, b in idx_refs(g, sl):
            pltpu.async_copy(a, b, isem)

    def drain_idx(g, sl):
        for a, b in idx_refs(g, sl):
            pltpu.make_async_copy(a, b, isem).wait()

    def issue_gathers(g, sl):
        for a, b in gather_refs(g, sl):
            pltpu.async_copy(a, b, gsem[sl])

    def drain_gathers(g, sl):
        for a, b in gather_refs(g, sl):
            pltpu.make_async_copy(a, b, gsem[sl]).wait()

    def issue_writes(g, sl):
        for a, b in write_refs(g, sl):
            pltpu.async_copy(a, b, wsem[sl])
        pltpu.sync_copy(co3, accw.at[dst3.at[sl]], add=True)

    def drain_writes(g, sl):
        for a, b in write_refs(g, sl):
            pltpu.make_async_copy(a, b, wsem[sl]).wait()

    def compute(sl):
        @plsc.parallel_loop(0, _C, unroll=4)
        def edge(j):
            zvec = jnp.zeros((16,), jnp.float32)
            for hl in range(4):
                o = hl * 16
                qv = q3[sl, j, pl.ds(o, 16)]
                kvv = kv3[sl, j, pl.ds(o, 16)]
                vv = kv3[sl, j, pl.ds(_HH + o, 16)]
                pev = pe3[sl, j, pl.ds(o, 16)]
                score = (kvv * qv) * pev
                pe3[sl, j, pl.ds(o, 16)] = score
                t = jnp.broadcast_to(jnp.sum(score), (16,))
                sv = jnp.exp(jnp.clip(t, -5.0, 5.0))
                co3[j, pl.ds(o, 16)] = vv * sv
                zvec = jnp.where(lane == hl, sv, zvec)
            co3[j, pl.ds(_HH, 16)] = zvec

    issue_idx(0, 0)
    drain_idx(0, 0)
    issue_idx(1, 1)
    drain_idx(1, 1)
    issue_gathers(0, 0)

    def step(g, sl):
        drain_gathers(g, sl)
        compute(sl)

        @pl.when(g >= 1)
        def _():
            drain_writes(g - 1, (sl + 2) % 3)

        issue_writes(g, sl)

        @pl.when(g <= _LAST - 1)
        def _():
            @pl.when(g >= 1)
            def _():
                drain_idx(g + 1, (sl + 1) % 3)

            issue_gathers(g + 1, (sl + 1) % 3)

        @pl.when(g <= _LAST - 2)
        def _():
            issue_idx(g + 2, (sl + 2) % 3)

    def triple(i, carry):
        g0 = i * 3
        step(g0, 0)
        step(g0 + 1, 1)
        step(g0 + 2, 2)
        return carry

    lax.fori_loop(0, _NCHUNK // 3, triple, 0)
    step(_NCHUNK - 1, (_NCHUNK - 1) % 3)
    drain_writes(_NCHUNK - 1, (_NCHUNK - 1) % 3)

    plsc.subcore_barrier()
    pltpu.sync_copy(accw.at[pl.ds(s * _RPS, _RPS)],
                    accw_out.at[c, pl.ds(s * _RPS, _RPS)])


_sc_attn = pl.kernel(
    _sc_attn_body,
    out_type=[
        jax.ShapeDtypeStruct((_E, _HD), jnp.float32),
        jax.ShapeDtypeStruct((_NC, _N, _AW), jnp.float32),
    ],
    mesh=plsc.VectorSubcoreMesh(core_axis_name="c", subcore_axis_name="s"),
    compiler_params=pltpu.CompilerParams(use_tc_tiling_on_sc=False,
                                         needs_layout_passes=False),
    scratch_types=[
        pltpu.VMEM((3, _C), jnp.int32),
        pltpu.VMEM((3, _C), jnp.int32),
        pltpu.VMEM((3, _C, _HH), jnp.float32),
        pltpu.VMEM((3, _C, 2 * _HH), jnp.float32),
        pltpu.VMEM((3, _C, _HH), jnp.float32),
        pltpu.VMEM((_C, _AW), jnp.float32),
        pltpu.VMEM_SHARED((_N, _AW), jnp.float32),
        pltpu.SemaphoreType.DMA,
        pltpu.SemaphoreType.DMA,
        pltpu.SemaphoreType.DMA,
        pltpu.SemaphoreType.DMA,
        pltpu.SemaphoreType.DMA,
        pltpu.SemaphoreType.DMA,
        pltpu.SemaphoreType.DMA,
    ],
)


_TB = 2000


def _tables_body(h_ref, wq_ref, bq_ref, wkv_ref, bkv_ref, q_out, kv_out):
    hb = h_ref[...]
    q_out[0] = jnp.dot(hb, wq_ref[0], preferred_element_type=jnp.float32,
                       precision=lax.Precision.HIGHEST) + bq_ref[0]
    kv_out[0] = jnp.dot(hb, wkv_ref[0], preferred_element_type=jnp.float32,
                        precision=lax.Precision.HIGHEST) + bkv_ref[0]


_tables = pl.pallas_call(
    _tables_body,
    grid=(_NC, _N // _TB),
    in_specs=[
        pl.BlockSpec((_TB, _HD), lambda cc, i: (i, 0)),
        pl.BlockSpec((1, _HD, _HH), lambda cc, i: (cc, 0, 0)),
        pl.BlockSpec((1, 1, _HH), lambda cc, i: (cc, 0, 0)),
        pl.BlockSpec((1, _HD, 2 * _HH), lambda cc, i: (cc, 0, 0)),
        pl.BlockSpec((1, 1, 2 * _HH), lambda cc, i: (cc, 0, 0)),
    ],
    out_specs=[
        pl.BlockSpec((1, _TB, _HH), lambda cc, i: (cc, i, 0)),
        pl.BlockSpec((1, _TB, 2 * _HH), lambda cc, i: (cc, i, 0)),
    ],
    out_shape=[
        jax.ShapeDtypeStruct((_NC, _N, _HH), jnp.float32),
        jax.ShapeDtypeStruct((_NC, _N, 2 * _HH), jnp.float32),
    ],
)

_EB = 2000


def _pe_body(e_ref, we_ref, be_ref, out_ref):
    out_ref[...] = (jnp.dot(e_ref[...], we_ref[...],
                            preferred_element_type=jnp.float32) * 0.25
                    + be_ref[...] * 0.25)


_pe = pl.pallas_call(
    _pe_body,
    grid=(_E // _EB,),
    in_specs=[
        pl.BlockSpec((_EB, _HD), lambda i: (i, 0)),
        pl.BlockSpec((_HD, _HD), lambda i: (0, 0)),
        pl.BlockSpec((1, _HD), lambda i: (0, 0)),
    ],
    out_specs=pl.BlockSpec((_EB, _HD), lambda i: (i, 0)),
    out_shape=jax.ShapeDtypeStruct((_E, _HD), jnp.float32),
)

_CB = 2000


def _combine_body(a0_ref, a1_ref, r_ref, o0_ref, o1_ref):
    zr0 = lax.dot_general(a0_ref[:, _HH:], r_ref[...],
                          (((1,), (0,)), ((), ())),
                          precision=lax.Precision.HIGHEST,
                          preferred_element_type=jnp.float32)
    zr1 = lax.dot_general(a1_ref[:, _HH:], r_ref[...],
                          (((1,), (0,)), ((), ())),
                          precision=lax.Precision.HIGHEST,
                          preferred_element_type=jnp.float32)
    o0_ref[...] = a0_ref[:, :_HH] / (zr0 + 1e-6)
    o1_ref[...] = a1_ref[:, :_HH] / (zr1 + 1e-6)


_combine = pl.pallas_call(
    _combine_body,
    grid=(_N // _CB,),
    in_specs=[
        pl.BlockSpec((_CB, _AW), lambda i: (i, 0)),
        pl.BlockSpec((_CB, _AW), lambda i: (i, 0)),
        pl.BlockSpec((16, _HH), lambda i: (0, 0)),
    ],
    out_specs=[
        pl.BlockSpec((_CB, _HH), lambda i: (i, 0)),
        pl.BlockSpec((_CB, _HH), lambda i: (i, 0)),
    ],
    out_shape=[
        jax.ShapeDtypeStruct((_N, _HH), jnp.float32),
        jax.ShapeDtypeStruct((_N, _HH), jnp.float32),
    ],
)


def kernel(edge_index, h, e, Wq, bq, Wk, bk, Wv, bv, We, be):
    src = edge_index[0].astype(jnp.int32)
    dst = edge_index[1].astype(jnp.int32)
    src2d = src.reshape(_E // _C, _C)
    dst2d = dst.reshape(_E // _C, _C)
    Wqs = jnp.stack([Wq[:, :_HH], Wq[:, _HH:]])
    bqs = jnp.stack([bq[:_HH], bq[_HH:]]).reshape(_NC, 1, _HH)
    Wkvs = jnp.stack([
        jnp.concatenate([Wk[:, :_HH], Wv[:, :_HH]], axis=1),
        jnp.concatenate([Wk[:, _HH:], Wv[:, _HH:]], axis=1),
    ])
    bkvs = jnp.stack([
        jnp.concatenate([bk[:_HH], bv[:_HH]]),
        jnp.concatenate([bk[_HH:], bv[_HH:]]),
    ]).reshape(_NC, 1, 2 * _HH)
    qtabs, kvtabs = _tables(h, Wqs, bqs, Wkvs, bkvs)
    pes = _pe(e, We, be.reshape(1, -1))
    eout3, accw = _sc_attn(src2d, dst2d, qtabs, kvtabs, pes)
    ra = jnp.concatenate([
        jnp.repeat(jnp.eye(4, dtype=jnp.float32), _D, axis=1),
        jnp.zeros((12, _HH), jnp.float32),
    ], axis=0)
    h0, h1 = _combine(accw[0], accw[1], ra)
    hout = jnp.concatenate([h0.reshape(_N, 4, _D), h1.reshape(_N, 4, _D)],
                           axis=1)
    return hout, eout3.reshape(_E, _H, _D)
